# baseline (speedup 1.0000x reference)
import functools
import jax
import jax.numpy as jnp
from jax import lax
from jax.experimental import pallas as pl
from jax.experimental.pallas import tpu as pltpu
from jax.experimental.pallas import tpu_sc as plsc

NEG_SLOPE = 0.2
N = 10000
NPAD = 10112
D = 128
DW = 144
NC = 2
NS = 16
NW = NC * NS
K = 64
CPB = 12
NBLK0 = 16
NBLK1 = 11
NCH0 = CPB * NBLK0
NCH1 = CPB * NBLK1
EPAD = NS * (NCH0 + NCH1) * K
RPS = NPAD // NS

_mesh = plsc.VectorSubcoreMesh(core_axis_name="c", subcore_axis_name="s")


def _dense_stats(h, asr, adr, st_s):
    asv = jnp.sum(h * asr, axis=1, keepdims=True)
    adv = jnp.sum(h * adr, axis=1, keepdims=True)
    rows = lax.broadcasted_iota(jnp.int32, (NPAD, 1), 0)
    valid = rows < N
    asv = jnp.where(valid, asv, -1e30)
    adv = jnp.where(valid, adv, -1e30)
    m = jnp.max(asv)
    t = adv + m
    sdv = jnp.maximum(t, NEG_SLOPE * t)
    st_s[:, 0:1] = asv
    st_s[:, 1:2] = adv
    st_s[:, 2:3] = sdv
    st_s[:, 3:16] = jnp.zeros((NPAD, 13), jnp.float32)


def _write_h144(h, h_s):
    h_s[:, 0:D] = h
    lane = lax.broadcasted_iota(jnp.int32, (NPAD, DW - D), 1)
    h_s[:, D:DW] = jnp.where(lane == 0, 1.0, 0.0)


def _dense1_body(x_ref, w_ref, asr_ref, adr_ref, h_s, st_s):
    h = jnp.dot(x_ref[:], w_ref[:], preferred_element_type=jnp.float32)
    hp = jnp.concatenate([h, jnp.zeros((NPAD - N, D), jnp.float32)], axis=0)
    _write_h144(hp, h_s)
    _dense_stats(hp, asr_ref[:], adr_ref[:], st_s)


def _comb_dense_body(num_ref, b_ref, w_ref, asr_ref, adr_ref, h_s, st_s):
    z = num_ref[0] + num_ref[1]
    x2 = jnp.maximum(z[:, 0:D] / (z[:, D:D + 1] + 1e-38) + b_ref[:], 0.0)
    h = jnp.dot(x2, w_ref[:], preferred_element_type=jnp.float32)
    _write_h144(h, h_s)
    _dense_stats(h, asr_ref[:], adr_ref[:], st_s)


def _final_body(num_ref, b_ref, wo_ref, bo_ref, out_ref):
    z = num_ref[0] + num_ref[1]
    x2 = jnp.maximum(z[:, 0:D] / (z[:, D:D + 1] + 1e-38) + b_ref[:], 0.0)
    logits = jnp.dot(x2, wo_ref[:], preferred_element_type=jnp.float32)
    logits = logits + bo_ref[:]
    lmax = jnp.max(logits, axis=1, keepdims=True)
    ex = jnp.exp(logits - lmax)
    out_ref[:] = ex / jnp.sum(ex, axis=1, keepdims=True)


def _sc_body(src2d, dst2d, h_hbm, st_hbm, num_out,
             sblk, dblk, sst_v, dstst_v, w_v, rows_v, acc,
             gsem, ssem, scsem):
    c = lax.axis_index("c")
    s = lax.axis_index("s")
    wid = s * NC + c
    zeros16 = jnp.zeros((16,), jnp.float32)

    @pl.loop(0, K)
    def _zrow(j):
        for m in range(DW // 16):
            rows_v[0, j, pl.ds(m * 16, 16)] = zeros16

    zbase = s * RPS
    for t in range(RPS // K):
        pltpu.sync_copy(rows_v.at[0], acc.at[pl.ds(zbase + t * K, K)])
    pltpu.sync_copy(rows_v.at[0, pl.ds(0, RPS % K)],
                    acc.at[pl.ds(zbase + (RPS // K) * K, RPS % K)])
    plsc.subcore_barrier()

    rowsel = jnp.arange(16, dtype=jnp.int32)
    col0 = jnp.zeros((16,), jnp.int32)
    rowbase = jnp.where(c == 0, s * NCH0, NS * NCH0 + s * NCH1)
    nblk = jnp.where(c == 0, NBLK0, NBLK1)

    def fire(j, bb):
        pltpu.async_copy(h_hbm.at[sblk.at[j]], rows_v.at[bb], gsem)
        pltpu.async_copy(st_hbm.at[sblk.at[j]], sst_v.at[bb], ssem)
        pltpu.async_copy(st_hbm.at[dblk.at[j]], dstst_v.at[bb], ssem)

    def drain_scatter():
        pltpu.make_async_copy(rows_v.at[0], acc.at[dblk.at[0]], scsem).wait()

    @pl.loop(0, nblk)
    def _block(blk):
        @pl.when(blk > 0)
        def _():
            drain_scatter()

        pltpu.sync_copy(src2d.at[pl.ds(rowbase + blk * CPB, CPB)], sblk)
        pltpu.sync_copy(dst2d.at[pl.ds(rowbase + blk * CPB, CPB)], dblk)
        fire(0, 0)
        fire(1, 1)

        @pl.loop(0, CPB // 3)
        def _tri(t):
            for b in range(3):
                j = t * 3 + b
                if b == 0:
                    @pl.when(t > 0)
                    def _():
                        drain_scatter()
                else:
                    drain_scatter()
                if b == 0:
                    fire(j + 2, 2)
                else:
                    @pl.when(t < CPB // 3 - 1)
                    def _():
                        fire(t * 3 + b + 2, (b + 2) % 3)
                sbj = b
                pltpu.make_async_copy(
                    st_hbm.at[sblk.at[0]], sst_v.at[sbj], ssem).wait()
                pltpu.make_async_copy(
                    st_hbm.at[sblk.at[0]], dstst_v.at[sbj], ssem).wait()
                for g in range(K // 16):
                    rg = rowsel + g * 16
                    a_s = plsc.load_gather(sst_v.at[sbj], [rg, col0])
                    a_d = plsc.load_gather(dstst_v.at[sbj], [rg, col0 + 1])
                    s_d = plsc.load_gather(dstst_v.at[sbj], [rg, col0 + 2])
                    e = a_s + a_d
                    e = jnp.maximum(e, NEG_SLOPE * e)
                    w_v[sbj, pl.ds(g * 16, 16)] = jnp.exp(e - s_d)
                rbj = b
                pltpu.make_async_copy(
                    h_hbm.at[sblk.at[0]], rows_v.at[rbj], gsem).wait()

                @plsc.parallel_loop(0, K // 16, unroll=4)
                def _scale(g):
                    wg = w_v[sbj, pl.ds(g * 16, 16)]
                    for r in range(16):
                        wj = wg[r]
                        jr = g * 16 + r
                        for m in range(DW // 16):
                            rows_v[rbj, jr, pl.ds(m * 16, 16)] = (
                                rows_v[rbj, jr, pl.ds(m * 16, 16)] * wj)

                pltpu.async_copy(rows_v.at[rbj], acc.at[dblk.at[j]],
                                 scsem, add=True)

    drain_scatter()

    plsc.subcore_barrier()
    for t in range(RPS // K):
        sl = pl.ds(zbase + t * K, K)
        pltpu.sync_copy(acc.at[sl], num_out.at[c, sl])
    sl = pl.ds(zbase + (RPS // K) * K, RPS % K)
    pltpu.sync_copy(acc.at[sl], num_out.at[c, sl])


_sc_edge = pl.kernel(
    _sc_body,
    out_type=jax.ShapeDtypeStruct((NC, NPAD, DW), jnp.float32),
    mesh=_mesh,
    compiler_params=pltpu.CompilerParams(
        needs_layout_passes=False, use_tc_tiling_on_sc=False),
    scratch_types=[
        pltpu.VMEM((CPB, K), jnp.int32),
        pltpu.VMEM((CPB, K), jnp.int32),
        pltpu.VMEM((3, K, 16), jnp.float32),
        pltpu.VMEM((3, K, 16), jnp.float32),
        pltpu.VMEM((3, K), jnp.float32),
        pltpu.VMEM((3, K, DW), jnp.float32),
        pltpu.VMEM_SHARED((NPAD, DW), jnp.float32),
        pltpu.SemaphoreType.DMA,
        pltpu.SemaphoreType.DMA,
        pltpu.SemaphoreType.DMA,
    ],
)


_HBM = pl.BlockSpec(memory_space=pltpu.MemorySpace.HBM)


def _dense1(x, W, asr, adr):
    return pl.pallas_call(
        _dense1_body,
        out_shape=(
            jax.ShapeDtypeStruct((NPAD, DW), jnp.float32),
            jax.ShapeDtypeStruct((NPAD, 16), jnp.float32),
        ),
    )(x, W, asr[None, :], adr[None, :])


def _comb_dense(num, b, W, asr, adr):
    return pl.pallas_call(
        _comb_dense_body,
        out_shape=(
            jax.ShapeDtypeStruct((NPAD, DW), jnp.float32),
            jax.ShapeDtypeStruct((NPAD, 16), jnp.float32),
        ),
    )(num, b[None, :], W, asr[None, :], adr[None, :])


def _final(num, b, Wo, bo):
    wo8 = jnp.zeros((D, 8), jnp.float32).at[:, :7].set(Wo)
    bo8 = jnp.full((8,), -1e30, jnp.float32).at[:7].set(bo)
    return pl.pallas_call(
        _final_body,
        out_shape=jax.ShapeDtypeStruct((NPAD, 8), jnp.float32),
    )(num, b[None, :], wo8, bo8[None, :])


import numpy as _np

_TAIL2D = _np.concatenate(
    [_np.arange(N, dtype=_np.int32),
     _np.full(EPAD - 320000 - N, NPAD - 1, dtype=_np.int32)]).reshape(-1, K)


def kernel(x, edge_index, W1, a_src1, a_dst1, b1, W2, a_src2, a_dst2, b2, Wo, bo):
    loop = jnp.arange(N, dtype=jnp.int32)
    pad = jnp.full((EPAD - (edge_index.shape[1] + N),), NPAD - 1, jnp.int32)
    srcp = jnp.concatenate([edge_index[0].astype(jnp.int32), loop, pad])
    dstp = jnp.concatenate([edge_index[1].astype(jnp.int32), loop, pad])
    h144, st = _dense1(x, W1, a_src1, a_dst1)
    src2d = srcp.reshape(EPAD // K, K)
    dst2d = dstp.reshape(EPAD // K, K)
    num = _sc_edge(src2d, dst2d, h144, st)
    h144, st = _comb_dense(num, b1, W2, a_src2, a_dst2)
    num = _sc_edge(src2d, dst2d, h144, st)
    out = _final(num, b2, Wo, bo)
    return out[:N, :7]


# speedup vs baseline: 1.2989x; 1.2989x over previous
import functools
import jax
import jax.numpy as jnp
from jax import lax
from jax.experimental import pallas as pl
from jax.experimental.pallas import tpu as pltpu
from jax.experimental.pallas import tpu_sc as plsc

NEG_SLOPE = 0.2
N = 10000
NPAD = 10112
D = 128
DW = 144
NC = 2
NS = 16
NW = NC * NS
K = 64
CPB = 12
NBLK0 = 16
NBLK1 = 11
NCH0 = CPB * NBLK0
NCH1 = CPB * NBLK1
EPAD = NS * (NCH0 + NCH1) * K
RPS = NPAD // NS

_mesh = plsc.VectorSubcoreMesh(core_axis_name="c", subcore_axis_name="s")


def _dense_stats(h, asr, adr, st_s):
    asv = jnp.sum(h * asr, axis=1, keepdims=True)
    adv = jnp.sum(h * adr, axis=1, keepdims=True)
    rows = lax.broadcasted_iota(jnp.int32, (NPAD, 1), 0)
    valid = rows < N
    asv = jnp.where(valid, asv, -1e30)
    adv = jnp.where(valid, adv, -1e30)
    m = jnp.max(asv)
    t = adv + m
    sdv = jnp.maximum(t, NEG_SLOPE * t)
    st_s[:, 0:1] = asv
    st_s[:, 1:2] = adv
    st_s[:, 2:3] = sdv
    st_s[:, 3:16] = jnp.zeros((NPAD, 13), jnp.float32)


def _write_h144(h, h_s):
    h_s[:, 0:D] = h
    lane = lax.broadcasted_iota(jnp.int32, (NPAD, DW - D), 1)
    h_s[:, D:DW] = jnp.where(lane == 0, 1.0, 0.0)


def _dense1_body(x_ref, w_ref, asr_ref, adr_ref, h_s, st_s):
    h = jnp.dot(x_ref[:], w_ref[:], preferred_element_type=jnp.float32)
    hp = jnp.concatenate([h, jnp.zeros((NPAD - N, D), jnp.float32)], axis=0)
    _write_h144(hp, h_s)
    _dense_stats(hp, asr_ref[:], adr_ref[:], st_s)


def _comb_dense_body(num_ref, b_ref, w_ref, asr_ref, adr_ref, h_s, st_s):
    z = num_ref[0] + num_ref[1]
    x2 = jnp.maximum(z[:, 0:D] / (z[:, D:D + 1] + 1e-38) + b_ref[:], 0.0)
    h = jnp.dot(x2, w_ref[:], preferred_element_type=jnp.float32)
    _write_h144(h, h_s)
    _dense_stats(h, asr_ref[:], adr_ref[:], st_s)


def _final_body(num_ref, b_ref, wo_ref, bo_ref, out_ref):
    z = num_ref[0] + num_ref[1]
    x2 = jnp.maximum(z[:, 0:D] / (z[:, D:D + 1] + 1e-38) + b_ref[:], 0.0)
    logits = jnp.dot(x2, wo_ref[:], preferred_element_type=jnp.float32)
    logits = logits + bo_ref[:]
    lmax = jnp.max(logits, axis=1, keepdims=True)
    ex = jnp.exp(logits - lmax)
    out_ref[:] = ex / jnp.sum(ex, axis=1, keepdims=True)


def _sc_body(src2d, dst2d, h_hbm, st_hbm, num_out,
             sblk, dblk, sst_v, dstst_v, w_v, rows_v, acc,
             gsem, ssem, scsem):
    c = lax.axis_index("c")
    s = lax.axis_index("s")
    wid = s * NC + c
    zeros16 = jnp.zeros((16,), jnp.float32)

    @pl.loop(0, K)
    def _zrow(j):
        for m in range(DW // 16):
            rows_v[0, j, pl.ds(m * 16, 16)] = zeros16

    zbase = s * RPS
    for t in range(RPS // K):
        pltpu.sync_copy(rows_v.at[0], acc.at[pl.ds(zbase + t * K, K)])
    pltpu.sync_copy(rows_v.at[0, pl.ds(0, RPS % K)],
                    acc.at[pl.ds(zbase + (RPS // K) * K, RPS % K)])
    plsc.subcore_barrier()

    rowsel = jnp.arange(16, dtype=jnp.int32)
    col0 = jnp.zeros((16,), jnp.int32)
    rowbase = jnp.where(c == 0, s * NCH0, NS * NCH0 + s * NCH1)
    nblk = jnp.where(c == 0, NBLK0, NBLK1)

    def fire(j, bb):
        pltpu.async_copy(h_hbm.at[sblk.at[j]], rows_v.at[bb], gsem)
        pltpu.async_copy(st_hbm.at[sblk.at[j]], sst_v.at[bb], ssem)
        pltpu.async_copy(st_hbm.at[dblk.at[j]], dstst_v.at[bb], ssem)

    def drain_scatter():
        pltpu.make_async_copy(rows_v.at[0], acc.at[dblk.at[0]], scsem).wait()

    @pl.loop(0, nblk)
    def _block(blk):
        @pl.when(blk > 0)
        def _():
            drain_scatter()

        pltpu.sync_copy(src2d.at[pl.ds(rowbase + blk * CPB, CPB)], sblk)
        pltpu.sync_copy(dst2d.at[pl.ds(rowbase + blk * CPB, CPB)], dblk)
        fire(0, 0)
        fire(1, 1)

        @pl.loop(0, CPB // 3)
        def _tri(t):
            for b in range(3):
                j = t * 3 + b
                if b == 0:
                    @pl.when(t > 0)
                    def _():
                        drain_scatter()
                else:
                    drain_scatter()
                if b == 0:
                    fire(j + 2, 2)
                else:
                    @pl.when(t < CPB // 3 - 1)
                    def _():
                        fire(t * 3 + b + 2, (b + 2) % 3)
                sbj = b
                pltpu.make_async_copy(
                    st_hbm.at[sblk.at[0]], sst_v.at[sbj], ssem).wait()
                pltpu.make_async_copy(
                    st_hbm.at[sblk.at[0]], dstst_v.at[sbj], ssem).wait()
                for g in range(K // 16):
                    rg = rowsel + g * 16
                    a_s = plsc.load_gather(sst_v.at[sbj], [rg, col0])
                    a_d = plsc.load_gather(dstst_v.at[sbj], [rg, col0 + 1])
                    s_d = plsc.load_gather(dstst_v.at[sbj], [rg, col0 + 2])
                    e = a_s + a_d
                    e = jnp.maximum(e, NEG_SLOPE * e)
                    w_v[sbj, pl.ds(g * 16, 16)] = jnp.exp(e - s_d)
                rbj = b
                pltpu.make_async_copy(
                    h_hbm.at[sblk.at[0]], rows_v.at[rbj], gsem).wait()

                for g in range(K // 16):
                    rg = rowsel + g * 16
                    plsc.store_scatter(rows_v.at[rbj], [rg, col0 + D],
                                       w_v[sbj, pl.ds(g * 16, 16)])

                @plsc.parallel_loop(0, K // 16, unroll=2)
                def _scale(g):
                    wg = w_v[sbj, pl.ds(g * 16, 16)]
                    for r in range(16):
                        wj = wg[r]
                        jr = g * 16 + r
                        for m in range(D // 16):
                            rows_v[rbj, jr, pl.ds(m * 16, 16)] = (
                                rows_v[rbj, jr, pl.ds(m * 16, 16)] * wj)

                pltpu.async_copy(rows_v.at[rbj], acc.at[dblk.at[j]],
                                 scsem, add=True)

    drain_scatter()

    plsc.subcore_barrier()
    for t in range(RPS // K):
        sl = pl.ds(zbase + t * K, K)
        pltpu.sync_copy(acc.at[sl], num_out.at[c, sl])
    sl = pl.ds(zbase + (RPS // K) * K, RPS % K)
    pltpu.sync_copy(acc.at[sl], num_out.at[c, sl])


_sc_edge = pl.kernel(
    _sc_body,
    out_type=jax.ShapeDtypeStruct((NC, NPAD, DW), jnp.float32),
    mesh=_mesh,
    compiler_params=pltpu.CompilerParams(
        needs_layout_passes=False, use_tc_tiling_on_sc=False),
    scratch_types=[
        pltpu.VMEM((CPB, K), jnp.int32),
        pltpu.VMEM((CPB, K), jnp.int32),
        pltpu.VMEM((3, K, 16), jnp.float32),
        pltpu.VMEM((3, K, 16), jnp.float32),
        pltpu.VMEM((3, K), jnp.float32),
        pltpu.VMEM((3, K, DW), jnp.float32),
        pltpu.VMEM_SHARED((NPAD, DW), jnp.float32),
        pltpu.SemaphoreType.DMA,
        pltpu.SemaphoreType.DMA,
        pltpu.SemaphoreType.DMA,
    ],
)


_HBM = pl.BlockSpec(memory_space=pltpu.MemorySpace.HBM)


def _dense1(x, W, asr, adr):
    return pl.pallas_call(
        _dense1_body,
        out_shape=(
            jax.ShapeDtypeStruct((NPAD, DW), jnp.float32),
            jax.ShapeDtypeStruct((NPAD, 16), jnp.float32),
        ),
    )(x, W, asr[None, :], adr[None, :])


def _comb_dense(num, b, W, asr, adr):
    return pl.pallas_call(
        _comb_dense_body,
        out_shape=(
            jax.ShapeDtypeStruct((NPAD, DW), jnp.float32),
            jax.ShapeDtypeStruct((NPAD, 16), jnp.float32),
        ),
    )(num, b[None, :], W, asr[None, :], adr[None, :])


def _final(num, b, Wo, bo):
    wo8 = jnp.zeros((D, 8), jnp.float32).at[:, :7].set(Wo)
    bo8 = jnp.full((8,), -1e30, jnp.float32).at[:7].set(bo)
    return pl.pallas_call(
        _final_body,
        out_shape=jax.ShapeDtypeStruct((NPAD, 8), jnp.float32),
    )(num, b[None, :], wo8, bo8[None, :])


import numpy as _np

_TAIL2D = _np.concatenate(
    [_np.arange(N, dtype=_np.int32),
     _np.full(EPAD - 320000 - N, NPAD - 1, dtype=_np.int32)]).reshape(-1, K)


def kernel(x, edge_index, W1, a_src1, a_dst1, b1, W2, a_src2, a_dst2, b2, Wo, bo):
    loop = jnp.arange(N, dtype=jnp.int32)
    pad = jnp.full((EPAD - (edge_index.shape[1] + N),), NPAD - 1, jnp.int32)
    srcp = jnp.concatenate([edge_index[0].astype(jnp.int32), loop, pad])
    dstp = jnp.concatenate([edge_index[1].astype(jnp.int32), loop, pad])
    h144, st = _dense1(x, W1, a_src1, a_dst1)
    src2d = srcp.reshape(EPAD // K, K)
    dst2d = dstp.reshape(EPAD // K, K)
    num = _sc_edge(src2d, dst2d, h144, st)
    h144, st = _comb_dense(num, b1, W2, a_src2, a_dst2)
    num = _sc_edge(src2d, dst2d, h144, st)
    out = _final(num, b2, Wo, bo)
    return out[:N, :7]
